# baseline (speedup 1.0000x reference)
import functools

import jax
import jax.numpy as jnp
from jax import lax
from jax.experimental import pallas as pl
from jax.experimental.pallas import tpu as pltpu
from jax.experimental.pallas import tpu_sc as plsc

NI = 4096
NJ = 50
D = 128
NC = 2
NS = 16
NW = NC * NS
CHUNK = NI // NW
NBUF = 7
STEADY = (NJ - NBUF) // NBUF * NBUF
REM = NJ - NBUF - STEADY


def _gather_body(xt_hbm, table_hbm, out_hbm, idx_v, rows, gsem, wsem):
    wid = lax.axis_index("s") * NC + lax.axis_index("c")
    i0 = wid * CHUNK

    pltpu.sync_copy(xt_hbm.at[:, pl.ds(i0, CHUNK)], idx_v)

    def start_gather(c, b):
        pltpu.async_copy(table_hbm.at[idx_v.at[c]], rows[b], gsem[b])

    def wait_gather(b):
        pltpu.make_async_copy(table_hbm.at[idx_v.at[0]], rows[b],
                              gsem[b]).wait()

    def start_wb(c, b):
        pltpu.async_copy(rows[b], out_hbm.at[c, pl.ds(i0, CHUNK)], wsem[b])

    def wait_wb(b):
        pltpu.make_async_copy(rows[b], out_hbm.at[0, pl.ds(0, CHUNK)],
                              wsem[b]).wait()

    for b in range(NBUF - 1):
        start_gather(b, b)

    wait_gather(0)
    start_wb(0, 0)
    start_gather(NBUF - 1, NBUF - 1)

    def steady_step(c, b, bprev):
        wait_gather(b)
        start_wb(c, b)
        wait_wb(bprev)
        start_gather(c + NBUF - 1, bprev)

    def outer(t, carry):
        c0 = 1 + NBUF * t
        for k in range(NBUF):
            steady_step(c0 + k, (1 + k) % NBUF, k)
        return carry

    lax.fori_loop(0, STEADY // NBUF, outer, 0)
    for k in range(REM):
        c = 1 + STEADY + k
        steady_step(c, c % NBUF, (c - 1) % NBUF)

    for k in range(NBUF - 1):
        c = NJ - (NBUF - 1) + k
        b = c % NBUF
        wait_gather(b)
        start_wb(c, b)
        wait_wb((c - 1) % NBUF)
    wait_wb((NJ - 1) % NBUF)


@jax.jit
def _gather(xt, table):
    mesh = plsc.VectorSubcoreMesh(core_axis_name="c", subcore_axis_name="s")
    f = functools.partial(
        pl.kernel,
        mesh=mesh,
        out_type=jax.ShapeDtypeStruct((NJ, NI, D), jnp.float32),
        scratch_types=[
            pltpu.VMEM((NJ, CHUNK), jnp.int32),
            [pltpu.VMEM((CHUNK, D), jnp.float32)] * NBUF,
            [pltpu.SemaphoreType.DMA] * NBUF,
            [pltpu.SemaphoreType.DMA] * NBUF,
        ],
        compiler_params=pltpu.CompilerParams(use_tc_tiling_on_sc=True),
    )(_gather_body)
    return f(xt, table)


def kernel(x, table):
    out_t = _gather(x.T.astype(jnp.int32), table)
    return out_t.transpose(1, 0, 2)


# speedup vs baseline: 1.0065x; 1.0065x over previous
import functools

import jax
import jax.numpy as jnp
from jax import lax
from jax.experimental import pallas as pl
from jax.experimental.pallas import tpu as pltpu
from jax.experimental.pallas import tpu_sc as plsc

NI = 4096
NJ = 50
D = 128
NC = 2
NS = 16
NW = NC * NS
CHUNK = NI // NW
NBUF = 5
STEADY = (NJ - NBUF) // NBUF * NBUF
REM = NJ - NBUF - STEADY


def _gather_body(xt_hbm, table_hbm, out_hbm, idx_v, rows, gsem, wsem):
    wid = lax.axis_index("s") * NC + lax.axis_index("c")
    i0 = wid * CHUNK

    pltpu.sync_copy(xt_hbm.at[:, pl.ds(i0, CHUNK)], idx_v)

    def start_gather(c, b):
        pltpu.async_copy(table_hbm.at[idx_v.at[c]], rows[b], gsem[b])

    def wait_gather(b):
        pltpu.make_async_copy(table_hbm.at[idx_v.at[0]], rows[b],
                              gsem[b]).wait()

    def start_wb(c, b):
        pltpu.async_copy(rows[b], out_hbm.at[c, pl.ds(i0, CHUNK)], wsem[b])

    def wait_wb(b):
        pltpu.make_async_copy(rows[b], out_hbm.at[0, pl.ds(0, CHUNK)],
                              wsem[b]).wait()

    for b in range(NBUF - 1):
        start_gather(b, b)

    wait_gather(0)
    start_wb(0, 0)
    start_gather(NBUF - 1, NBUF - 1)

    def steady_step(c, b, bprev):
        wait_gather(b)
        start_wb(c, b)
        wait_wb(bprev)
        start_gather(c + NBUF - 1, bprev)

    def outer(t, carry):
        c0 = 1 + NBUF * t
        for k in range(NBUF):
            steady_step(c0 + k, (1 + k) % NBUF, k)
        return carry

    lax.fori_loop(0, STEADY // NBUF, outer, 0)
    for k in range(REM):
        c = 1 + STEADY + k
        steady_step(c, c % NBUF, (c - 1) % NBUF)

    for k in range(NBUF - 1):
        c = NJ - (NBUF - 1) + k
        b = c % NBUF
        wait_gather(b)
        start_wb(c, b)
        wait_wb((c - 1) % NBUF)
    wait_wb((NJ - 1) % NBUF)


@jax.jit
def _gather(xt, table):
    mesh = plsc.VectorSubcoreMesh(core_axis_name="c", subcore_axis_name="s")
    f = functools.partial(
        pl.kernel,
        mesh=mesh,
        out_type=jax.ShapeDtypeStruct((NJ, NI, D), jnp.float32),
        scratch_types=[
            pltpu.VMEM((NJ, CHUNK), jnp.int32),
            [pltpu.VMEM((CHUNK, D), jnp.float32)] * NBUF,
            [pltpu.SemaphoreType.DMA] * NBUF,
            [pltpu.SemaphoreType.DMA] * NBUF,
        ],
        compiler_params=pltpu.CompilerParams(use_tc_tiling_on_sc=True),
    )(_gather_body)
    return f(xt, table)


def kernel(x, table):
    out_t = _gather(x.T.astype(jnp.int32), table)
    return out_t.transpose(1, 0, 2)
